# baseline (speedup 1.0000x reference)
import jax
import jax.numpy as jnp
from jax.experimental import pallas as pl
from jax.experimental.pallas import tpu as pltpu

_NP = 768
_CHUNK = 32
_MAX_CHUNKS = 8


def _moe_body(start_ref, nch_ref, g_ref, p_ref, x_ref, w13_ref, w2_ref,
              out_ref, xs_ref, os_ref):
    e = pl.program_id(0)
    nexp = pl.num_programs(0)

    @pl.when(e == 0)
    def _init():
        os_ref[...] = jnp.zeros_like(os_ref)
        xs_ref[...] = jax.lax.dot_general(
            g_ref[...], x_ref[...], (((1,), (0,)), ((), ())),
            preferred_element_type=jnp.float32).astype(jnp.bfloat16)

    w13 = w13_ref[0].astype(jnp.bfloat16)
    w2 = w2_ref[0].astype(jnp.bfloat16)
    inter = w2.shape[1]
    base = start_ref[e]

    for c in range(1):
        @pl.when(c < nch_ref[e])
        def _chunk(c=c):
            row = pl.multiple_of(base + c * _CHUNK, 8)
            xc = xs_ref[pl.ds(0, 128), :]
            gu = jax.lax.dot_general(
                xc, w13, (((1,), (1,)), ((), ())),
                preferred_element_type=jnp.float32)
            gate = gu[:, :inter]
            up = gu[:, inter:]
            h = (gate * jax.nn.sigmoid(gate) * up).astype(jnp.bfloat16)
            o = jax.lax.dot_general(
                h, w2, (((1,), (1,)), ((), ())),
                preferred_element_type=jnp.float32)
            os_ref[pl.ds(row, _CHUNK), :] = o[:_CHUNK]

    @pl.when(e == nexp - 1)
    def _combine():
        out_ref[...] = jax.lax.dot_general(
            p_ref[...], os_ref[...], (((1,), (0,)), ((), ())),
            preferred_element_type=jnp.float32,
            precision=jax.lax.Precision.HIGHEST)


def kernel(hidden_states, topk_ids, topk_weights, w13, w2):
    tokens, hidden = hidden_states.shape
    num_experts, two_inter, _ = w13.shape
    inter = w2.shape[2]
    topk = topk_ids.shape[1]
    nslots = tokens * topk

    ids32 = topk_ids.astype(jnp.int32)
    wts = topk_weights.astype(jnp.float32)
    x16 = hidden_states.astype(jnp.bfloat16)

    eid = ids32.reshape(-1)
    wtv = wts.reshape(-1)
    tokv = (jnp.arange(nslots, dtype=jnp.int32) // topk)
    oh = (eid[:, None] == jnp.arange(num_experts, dtype=jnp.int32)[None, :])
    ohi = oh.astype(jnp.int32)
    cnt = ohi.sum(axis=0)
    pad8 = ((cnt + 7) // 8) * 8
    start = (jnp.cumsum(pad8) - pad8).astype(jnp.int32)
    nch = ((cnt + _CHUNK - 1) // _CHUNK).astype(jnp.int32)
    rank = jnp.sum((jnp.cumsum(ohi, axis=0) - 1) * ohi, axis=1)
    pos = jnp.sum(ohi * start[None, :], axis=1) + rank
    sorted_tok = jnp.zeros(_NP, jnp.int32).at[pos].set(tokv)
    sorted_wt = jnp.zeros(_NP, jnp.float32).at[pos].set(wtv)
    gmat = (sorted_tok[:, None] == jnp.arange(tokens)[None, :]
            ).astype(jnp.bfloat16)
    pmat = ((jnp.arange(tokens)[:, None] == sorted_tok[None, :])
            .astype(jnp.float32) * sorted_wt[None, :])

    out = pl.pallas_call(
        _moe_body,
        grid_spec=pltpu.PrefetchScalarGridSpec(
            num_scalar_prefetch=2,
            grid=(num_experts,),
            in_specs=[
                pl.BlockSpec((_NP, tokens), lambda e, *_: (0, 0)),
                pl.BlockSpec((tokens, _NP), lambda e, *_: (0, 0)),
                pl.BlockSpec((tokens, hidden), lambda e, *_: (0, 0)),
                pl.BlockSpec((1, two_inter, hidden), lambda e, *_: (e, 0, 0)),
                pl.BlockSpec((1, hidden, inter), lambda e, *_: (e, 0, 0)),
            ],
            out_specs=pl.BlockSpec((tokens, hidden), lambda e, *_: (0, 0)),
            scratch_shapes=[
                pltpu.VMEM((_NP, hidden), jnp.bfloat16),
                pltpu.VMEM((_NP, hidden), jnp.float32),
            ],
        ),
        out_shape=jax.ShapeDtypeStruct((tokens, hidden), jnp.float32),
    )(start, nch, gmat, pmat, x16, w13, w2)
    return out


# speedup vs baseline: 1.0846x; 1.0846x over previous
import jax
import jax.numpy as jnp
from jax.experimental import pallas as pl
from jax.experimental.pallas import tpu as pltpu

_NP = 1280
_CHUNK = 32
_MAX_CHUNKS = 8
_ALIGN = 16


def _prep_body(ids_ref, wts_ref, start_ref, nch_ref, gt_ref, p_ref):
    ids = ids_ref[...]
    wts = wts_ref[...]
    tokens = ids.shape[0]
    nexp = start_ref.shape[1]

    ecols = jax.lax.broadcasted_iota(jnp.int32, (tokens, nexp), 1)
    oh0 = (ids[:, 0:1] == ecols).astype(jnp.float32)
    oh1 = (ids[:, 1:2] == ecols).astype(jnp.float32)
    ctok = oh0 + oh1

    rr = jax.lax.broadcasted_iota(jnp.int32, (tokens, tokens), 0)
    cc = jax.lax.broadcasted_iota(jnp.int32, (tokens, tokens), 1)
    tril = (rr > cc).astype(jnp.float32)
    count_lt = jax.lax.dot_general(
        tril, ctok, (((1,), (0,)), ((), ())),
        preferred_element_type=jnp.float32)

    cnt = jnp.sum(ctok, axis=0, keepdims=True)
    cnt_i = cnt.astype(jnp.int32)
    pad = (((cnt_i + _ALIGN - 1) // _ALIGN) * _ALIGN).astype(jnp.float32)
    er = jax.lax.broadcasted_iota(jnp.int32, (nexp, nexp), 0)
    ec = jax.lax.broadcasted_iota(jnp.int32, (nexp, nexp), 1)
    before = (er < ec).astype(jnp.float32)
    start = jax.lax.dot_general(
        pad, before, (((1,), (0,)), ((), ())),
        preferred_element_type=jnp.float32)

    rank0 = jnp.sum(count_lt * oh0, axis=1, keepdims=True)
    rank1 = (jnp.sum(count_lt * oh1, axis=1, keepdims=True)
             + (ids[:, 0:1] == ids[:, 1:2]).astype(jnp.float32))
    sat0 = jnp.sum(oh0 * start, axis=1, keepdims=True)
    sat1 = jnp.sum(oh1 * start, axis=1, keepdims=True)
    pos0 = sat0 + rank0
    pos1 = sat1 + rank1

    piota = jax.lax.broadcasted_iota(jnp.int32, (tokens, _NP), 1)
    is0 = (piota == pos0.astype(jnp.int32)).astype(jnp.float32)
    is1 = (piota == pos1.astype(jnp.int32)).astype(jnp.float32)
    gt_ref[...] = (is0 + is1).astype(jnp.bfloat16)
    p_ref[...] = wts[:, 0:1] * is0 + wts[:, 1:2] * is1

    start_ref[...] = start.astype(jnp.int32)
    nch_ref[...] = (cnt_i + _CHUNK - 1) // _CHUNK


def _moe_body(start_ref, nch_ref, gt_ref, p_ref, x_ref, w13_ref, w2_ref,
              out_ref, xs_ref, os_ref):
    e = pl.program_id(0)
    nexp = pl.num_programs(0)

    @pl.when(e == 0)
    def _init():
        os_ref[...] = jnp.zeros_like(os_ref)
        xs_ref[...] = jax.lax.dot_general(
            gt_ref[...], x_ref[...], (((0,), (0,)), ((), ())),
            preferred_element_type=jnp.float32).astype(jnp.bfloat16)

    w13 = w13_ref[0].astype(jnp.bfloat16)
    w2 = w2_ref[0].astype(jnp.bfloat16)
    inter = w2.shape[1]
    base = start_ref[0, e]

    for c in range(_MAX_CHUNKS):
        @pl.when(c < nch_ref[0, e])
        def _chunk(c=c):
            row = pl.multiple_of(base + c * _CHUNK, _ALIGN)
            xc = xs_ref[pl.ds(row, _CHUNK), :]
            gu = jax.lax.dot_general(
                xc, w13, (((1,), (1,)), ((), ())),
                preferred_element_type=jnp.float32)
            gate = gu[:, :inter]
            up = gu[:, inter:]
            h = (gate * jax.nn.sigmoid(gate) * up).astype(jnp.bfloat16)
            o = jax.lax.dot_general(
                h, w2, (((1,), (1,)), ((), ())),
                preferred_element_type=jnp.float32)
            os_ref[pl.ds(row, _CHUNK), :] = o

    @pl.when(e == nexp - 1)
    def _combine():
        out_ref[...] = jax.lax.dot_general(
            p_ref[...], os_ref[...], (((1,), (0,)), ((), ())),
            preferred_element_type=jnp.float32,
            precision=jax.lax.Precision.HIGHEST)


def kernel(hidden_states, topk_ids, topk_weights, w13, w2):
    tokens, hidden = hidden_states.shape
    num_experts, two_inter, _ = w13.shape
    inter = w2.shape[2]

    ids32 = topk_ids.astype(jnp.int32)
    wts = topk_weights.astype(jnp.float32)
    x16 = hidden_states.astype(jnp.bfloat16)

    start, nch, gt, pmat = pl.pallas_call(
        _prep_body,
        out_shape=(
            jax.ShapeDtypeStruct((1, num_experts), jnp.int32),
            jax.ShapeDtypeStruct((1, num_experts), jnp.int32),
            jax.ShapeDtypeStruct((tokens, _NP), jnp.bfloat16),
            jax.ShapeDtypeStruct((tokens, _NP), jnp.float32),
        ),
    )(ids32, wts)

    out = pl.pallas_call(
        _moe_body,
        grid_spec=pltpu.PrefetchScalarGridSpec(
            num_scalar_prefetch=2,
            grid=(num_experts,),
            in_specs=[
                pl.BlockSpec((tokens, _NP), lambda e, *_: (0, 0)),
                pl.BlockSpec((tokens, _NP), lambda e, *_: (0, 0)),
                pl.BlockSpec((tokens, hidden), lambda e, *_: (0, 0)),
                pl.BlockSpec((1, two_inter, hidden), lambda e, *_: (e, 0, 0)),
                pl.BlockSpec((1, hidden, inter), lambda e, *_: (e, 0, 0)),
            ],
            out_specs=pl.BlockSpec((tokens, hidden), lambda e, *_: (0, 0)),
            scratch_shapes=[
                pltpu.VMEM((_NP, hidden), jnp.bfloat16),
                pltpu.VMEM((_NP, hidden), jnp.float32),
            ],
        ),
        out_shape=jax.ShapeDtypeStruct((tokens, hidden), jnp.float32),
    )(start, nch, gt, pmat, x16, w13, w2)
    return out
